# baseline (speedup 1.0000x reference)
import functools

import jax
import jax.numpy as jnp
from jax import lax
from jax.experimental import pallas as pl
from jax.experimental.pallas import tpu as pltpu
from jax.experimental.pallas import tpu_sc as plsc

_NUM_CLASSES = 285
_PAD_CLASSES = 384
_BINS = 288
_LANE_STRIDE = 289
_BIG = 2**30
_L = 16


def _tc_body(x_ref, w_ref, b_ref, o_ref):
    logits = jnp.dot(x_ref[...], w_ref[...],
                     preferred_element_type=jnp.float32) + b_ref[...]
    tt = logits.shape[0]
    cls = jax.lax.broadcasted_iota(jnp.int32, (tt, _PAD_CLASSES), 1)
    m = jnp.max(logits, axis=1, keepdims=True)
    votes = jnp.min(jnp.where(logits == m, cls, _BIG), axis=1, keepdims=True)
    o_ref[...] = votes.reshape(tt // 128, 128)


def _sc_hist_body(votes_hbm, out_hbm, votes_v, hist_pl, onehot_v, comb_v,
                  spm, *, per_w):
    sid = lax.axis_index("s")
    n_chunks = per_w // _L
    zeros16 = jnp.zeros((_L,), jnp.float32)
    ones16 = jnp.ones((_L,), jnp.float32)
    lane = jax.lax.broadcasted_iota(jnp.int32, (_L,), 0)
    lane_base = lane * _LANE_STRIDE

    pltpu.sync_copy(votes_hbm.at[pl.ds(sid * per_w, per_w)], votes_v)

    for k in range(_LANE_STRIDE):
        hist_pl[pl.ds(k * _L, _L)] = zeros16
    for i in range(n_chunks):
        idx = votes_v[pl.ds(i * _L, _L)] + lane_base
        plsc.addupdate_scatter(hist_pl, [idx], ones16)

    for j in range(_BINS // _L):
        acc = zeros16
        for l in range(_L):
            acc = acc + hist_pl[pl.ds(l * _LANE_STRIDE + j * _L, _L)]
        onehot_v[pl.ds(j * _L, _L)] = acc
    pltpu.sync_copy(onehot_v, spm.at[sid])
    plsc.subcore_barrier()

    @pl.when(sid == 0)
    def _():
        pltpu.sync_copy(spm, comb_v)
        n_sub = spm.shape[0]
        chunk_max = jnp.full((_L,), -1.0, jnp.float32)
        for j in range(_BINS // _L):
            acc = zeros16
            for l in range(n_sub):
                acc = acc + comb_v[l, pl.ds(j * _L, _L)]
            onehot_v[pl.ds(j * _L, _L)] = acc
            chunk_max = jnp.maximum(chunk_max, acc)
        hm = jnp.max(chunk_max, axis=0)
        winner = jnp.int32(_BIG)
        bins16 = jax.lax.broadcasted_iota(jnp.int32, (_L,), 0)
        for j in range(_BINS // _L):
            v = onehot_v[pl.ds(j * _L, _L)]
            cand = jnp.min(jnp.where(v == hm, bins16 + j * _L, _BIG), axis=0)
            winner = jnp.minimum(winner, cand)
        for j in range(_BINS // _L):
            onehot_v[pl.ds(j * _L, _L)] = (bins16 + j * _L == winner).astype(
                jnp.float32)
        pltpu.sync_copy(onehot_v, out_hbm)


def _sc_hist(votes_flat):
    n_sub = plsc.get_sparse_core_info().num_subcores
    per_w = votes_flat.shape[0] // n_sub
    mesh = plsc.VectorSubcoreMesh(core_axis_name="c", subcore_axis_name="s",
                                  num_cores=1)
    return pl.kernel(
        functools.partial(_sc_hist_body, per_w=per_w),
        mesh=mesh,
        out_type=jax.ShapeDtypeStruct((_BINS,), jnp.float32),
        scratch_types=[
            pltpu.VMEM((per_w,), jnp.int32),
            pltpu.VMEM((_L * _LANE_STRIDE,), jnp.float32),
            pltpu.VMEM((_BINS,), jnp.float32),
            pltpu.VMEM((n_sub, _BINS), jnp.float32),
            pltpu.VMEM_SHARED((n_sub, _BINS), jnp.float32),
        ],
        compiler_params=pltpu.CompilerParams(needs_layout_passes=False),
    )(votes_flat)


def kernel(x, W, b):
    _, T, C = x.shape
    xr = x[0]
    w_pad = jnp.zeros((C, _PAD_CLASSES), jnp.float32).at[:, :_NUM_CLASSES].set(W.T)
    b_pad = jnp.full((1, _PAD_CLASSES), -3.4e38, jnp.float32).at[0, :_NUM_CLASSES].set(b)

    tt = 2048
    n_steps = T // tt
    votes = pl.pallas_call(
        _tc_body,
        grid=(n_steps,),
        in_specs=[
            pl.BlockSpec((tt, C), lambda i: (i, 0)),
            pl.BlockSpec((C, _PAD_CLASSES), lambda i: (0, 0)),
            pl.BlockSpec((1, _PAD_CLASSES), lambda i: (0, 0)),
        ],
        out_specs=pl.BlockSpec((tt // 128, 128), lambda i: (i, 0)),
        out_shape=jax.ShapeDtypeStruct((T // 128, 128), jnp.int32),
    )(xr, w_pad, b_pad).reshape(T)

    hist_onehot = _sc_hist(votes)
    return hist_onehot[None, :_NUM_CLASSES]


# speedup vs baseline: 1.0415x; 1.0415x over previous
import functools

import jax
import jax.numpy as jnp
from jax import lax
from jax.experimental import pallas as pl
from jax.experimental.pallas import tpu as pltpu
from jax.experimental.pallas import tpu_sc as plsc

_NUM_CLASSES = 285
_PAD_CLASSES = 384
_BINS = 288
_LANE_STRIDE = 289
_BIG = 2**30
_L = 16


def _tc_body(x_ref, w_ref, b_ref, o_ref):
    logits = jax.lax.dot_general(
        x_ref[...], w_ref[...],
        dimension_numbers=(((1,), (1,)), ((), ())),
        preferred_element_type=jnp.float32) + b_ref[...]
    tt, n_cls = logits.shape
    cls = jax.lax.broadcasted_iota(jnp.int32, (tt, n_cls), 1)
    m = jnp.max(logits, axis=1, keepdims=True)
    votes = jnp.min(jnp.where(logits == m, cls, _BIG), axis=1, keepdims=True)
    o_ref[...] = votes.reshape(tt // 128, 128)


def _sc_hist_body(votes_hbm, out_hbm, votes_v, hist_pl, onehot_v, comb_v,
                  spm, *, per_w):
    sid = lax.axis_index("s")
    n_chunks = per_w // _L
    zeros16 = jnp.zeros((_L,), jnp.float32)
    ones16 = jnp.ones((_L,), jnp.float32)
    lane = jax.lax.broadcasted_iota(jnp.int32, (_L,), 0)
    lane_base = lane * _LANE_STRIDE

    pltpu.sync_copy(votes_hbm.at[pl.ds(sid * per_w, per_w)], votes_v)

    for k in range(_LANE_STRIDE):
        hist_pl[pl.ds(k * _L, _L)] = zeros16
    for i in range(n_chunks):
        idx = votes_v[pl.ds(i * _L, _L)] + lane_base
        plsc.addupdate_scatter(hist_pl, [idx], ones16)

    for j in range(_BINS // _L):
        acc = zeros16
        for l in range(_L):
            acc = acc + hist_pl[pl.ds(l * _LANE_STRIDE + j * _L, _L)]
        onehot_v[pl.ds(j * _L, _L)] = acc
    pltpu.sync_copy(onehot_v, spm.at[sid])
    plsc.subcore_barrier()

    @pl.when(sid == 0)
    def _():
        pltpu.sync_copy(spm, comb_v)
        n_sub = spm.shape[0]
        chunk_max = jnp.full((_L,), -1.0, jnp.float32)
        for j in range(_BINS // _L):
            acc = zeros16
            for l in range(n_sub):
                acc = acc + comb_v[l, pl.ds(j * _L, _L)]
            onehot_v[pl.ds(j * _L, _L)] = acc
            chunk_max = jnp.maximum(chunk_max, acc)
        hm = jnp.max(chunk_max, axis=0)
        winner = jnp.int32(_BIG)
        bins16 = jax.lax.broadcasted_iota(jnp.int32, (_L,), 0)
        for j in range(_BINS // _L):
            v = onehot_v[pl.ds(j * _L, _L)]
            cand = jnp.min(jnp.where(v == hm, bins16 + j * _L, _BIG), axis=0)
            winner = jnp.minimum(winner, cand)
        for j in range(_BINS // _L):
            onehot_v[pl.ds(j * _L, _L)] = (bins16 + j * _L == winner).astype(
                jnp.float32)
        pltpu.sync_copy(onehot_v, out_hbm)


def _sc_hist(votes_flat):
    n_sub = plsc.get_sparse_core_info().num_subcores
    per_w = votes_flat.shape[0] // n_sub
    mesh = plsc.VectorSubcoreMesh(core_axis_name="c", subcore_axis_name="s",
                                  num_cores=1)
    return pl.kernel(
        functools.partial(_sc_hist_body, per_w=per_w),
        mesh=mesh,
        out_type=jax.ShapeDtypeStruct((_BINS,), jnp.float32),
        scratch_types=[
            pltpu.VMEM((per_w,), jnp.int32),
            pltpu.VMEM((_L * _LANE_STRIDE,), jnp.float32),
            pltpu.VMEM((_BINS,), jnp.float32),
            pltpu.VMEM((n_sub, _BINS), jnp.float32),
            pltpu.VMEM_SHARED((n_sub, _BINS), jnp.float32),
        ],
        compiler_params=pltpu.CompilerParams(needs_layout_passes=False),
    )(votes_flat)


def kernel(x, W, b):
    _, T, C = x.shape
    xr = x[0]
    n_cls = W.shape[0]

    tt = 2048
    n_steps = T // tt
    votes = pl.pallas_call(
        _tc_body,
        grid=(n_steps,),
        in_specs=[
            pl.BlockSpec((tt, C), lambda i: (i, 0)),
            pl.BlockSpec((n_cls, C), lambda i: (0, 0)),
            pl.BlockSpec((1, n_cls), lambda i: (0, 0)),
        ],
        out_specs=pl.BlockSpec((tt // 128, 128), lambda i: (i, 0)),
        out_shape=jax.ShapeDtypeStruct((T // 128, 128), jnp.int32),
    )(xr, W, b[None, :]).reshape(T)

    hist_onehot = _sc_hist(votes)
    return hist_onehot[None, :_NUM_CLASSES]


# speedup vs baseline: 1.0482x; 1.0064x over previous
import functools

import jax
import jax.numpy as jnp
from jax import lax
from jax.experimental import pallas as pl
from jax.experimental.pallas import tpu as pltpu
from jax.experimental.pallas import tpu_sc as plsc

_NUM_CLASSES = 285
_PAD_CLASSES = 384
_BINS = 288
_LANE_STRIDE = 289
_BIG = 2**30
_L = 16


def _tc_body(x_ref, w_ref, b_ref, o_ref):
    logits = jax.lax.dot_general(
        x_ref[...], w_ref[...],
        dimension_numbers=(((1,), (1,)), ((), ())),
        preferred_element_type=jnp.float32) + b_ref[...]
    tt, n_cls = logits.shape
    cls = jax.lax.broadcasted_iota(jnp.int32, (tt, n_cls), 1)
    m = jnp.max(logits, axis=1, keepdims=True)
    votes = jnp.min(jnp.where(logits == m, cls, _BIG), axis=1, keepdims=True)
    o_ref[...] = votes.reshape(tt // 128, 128)


def _sc_hist_body(votes_hbm, out_hbm, votes_v, hist_pl, onehot_v, comb_v,
                  spm, *, per_w):
    sid = lax.axis_index("s")
    n_chunks = per_w // _L
    zeros16 = jnp.zeros((_L,), jnp.float32)
    ones16 = jnp.ones((_L,), jnp.float32)
    lane = jax.lax.broadcasted_iota(jnp.int32, (_L,), 0)
    lane_base = lane * _LANE_STRIDE

    pltpu.sync_copy(votes_hbm.at[pl.ds(sid * per_w, per_w)], votes_v)

    def zero_body(k, _):
        hist_pl[pl.ds(k * _L, _L)] = zeros16
        return 0

    lax.fori_loop(0, _LANE_STRIDE, zero_body, 0)

    def scat_body(i, _):
        idx = votes_v[pl.ds(i * _L, _L)] + lane_base
        plsc.addupdate_scatter(hist_pl, [idx], ones16)
        return 0

    lax.fori_loop(0, n_chunks, scat_body, 0)

    def fold_body(j, _):
        def fold_inner(l, acc):
            return acc + hist_pl[pl.ds(l * _LANE_STRIDE + j * _L, _L)]

        onehot_v[pl.ds(j * _L, _L)] = lax.fori_loop(0, _L, fold_inner, zeros16)
        return 0

    lax.fori_loop(0, _BINS // _L, fold_body, 0)
    pltpu.sync_copy(onehot_v, spm.at[sid])
    plsc.subcore_barrier()

    @pl.when(sid == 0)
    def _():
        pltpu.sync_copy(spm, comb_v)
        n_sub = spm.shape[0]
        chunk_max = jnp.full((_L,), -1.0, jnp.float32)
        for j in range(_BINS // _L):
            acc = zeros16
            for l in range(n_sub):
                acc = acc + comb_v[l, pl.ds(j * _L, _L)]
            onehot_v[pl.ds(j * _L, _L)] = acc
            chunk_max = jnp.maximum(chunk_max, acc)
        hm = jnp.max(chunk_max, axis=0)
        winner = jnp.int32(_BIG)
        bins16 = jax.lax.broadcasted_iota(jnp.int32, (_L,), 0)
        for j in range(_BINS // _L):
            v = onehot_v[pl.ds(j * _L, _L)]
            cand = jnp.min(jnp.where(v == hm, bins16 + j * _L, _BIG), axis=0)
            winner = jnp.minimum(winner, cand)
        for j in range(_BINS // _L):
            onehot_v[pl.ds(j * _L, _L)] = (bins16 + j * _L == winner).astype(
                jnp.float32)
        pltpu.sync_copy(onehot_v.at[pl.ds(0, _NUM_CLASSES)], out_hbm)


def _sc_hist(votes_flat):
    n_sub = plsc.get_sparse_core_info().num_subcores
    per_w = votes_flat.shape[0] // n_sub
    mesh = plsc.VectorSubcoreMesh(core_axis_name="c", subcore_axis_name="s",
                                  num_cores=1)
    return pl.kernel(
        functools.partial(_sc_hist_body, per_w=per_w),
        mesh=mesh,
        out_type=jax.ShapeDtypeStruct((_NUM_CLASSES,), jnp.float32),
        scratch_types=[
            pltpu.VMEM((per_w,), jnp.int32),
            pltpu.VMEM((_L * _LANE_STRIDE,), jnp.float32),
            pltpu.VMEM((_BINS,), jnp.float32),
            pltpu.VMEM((n_sub, _BINS), jnp.float32),
            pltpu.VMEM_SHARED((n_sub, _BINS), jnp.float32),
        ],
        compiler_params=pltpu.CompilerParams(needs_layout_passes=False),
    )(votes_flat)


def kernel(x, W, b):
    _, T, C = x.shape
    xr = x[0]
    n_cls = W.shape[0]

    tt = 2048
    n_steps = T // tt
    votes = pl.pallas_call(
        _tc_body,
        grid=(n_steps,),
        in_specs=[
            pl.BlockSpec((tt, C), lambda i: (i, 0)),
            pl.BlockSpec((n_cls, C), lambda i: (0, 0)),
            pl.BlockSpec((1, n_cls), lambda i: (0, 0)),
        ],
        out_specs=pl.BlockSpec((tt // 128, 128), lambda i: (i, 0)),
        out_shape=jax.ShapeDtypeStruct((T // 128, 128), jnp.int32),
    )(xr, W, b[None, :]).reshape(T)

    hist_onehot = _sc_hist(votes)
    return hist_onehot[None, :]
